# baseline (speedup 1.0000x reference)
import functools

import jax
import jax.numpy as jnp
from jax import lax
from jax.experimental import pallas as pl
from jax.experimental.pallas import tpu as pltpu
from jax.experimental.pallas import tpu_sc as plsc

B, T, D = 4096, 200, 64
V = 1_000_000
ROWS = B * T
NC, NS, L = 2, 16, 16
NW = NC * NS
PAIRS = V // 2
ROWS_PER_W = ROWS // NW
CHUNK = 128
NCH = ROWS_PER_W // CHUNK
IDX_ROWS = ROWS // CHUNK
NBUF = 3

_mesh = plsc.VectorSubcoreMesh(core_axis_name="c", subcore_axis_name="s")


@functools.partial(
    pl.kernel,
    out_type=jax.ShapeDtypeStruct((ROWS // 2, 2 * D), jnp.float32),
    mesh=_mesh,
    scratch_types=[
        pltpu.VMEM((NCH, CHUNK), jnp.int32),
        pltpu.VMEM((NBUF, CHUNK), jnp.int32),
        pltpu.VMEM((NBUF, CHUNK, 2 * D), jnp.float32),
        pltpu.VMEM((NBUF, CHUNK // 2, 2 * D), jnp.float32),
        pltpu.VMEM((T, D), jnp.float32),
        pltpu.SemaphoreType.DMA((NBUF,)),
        pltpu.SemaphoreType.DMA((NBUF,)),
    ],
    compiler_params=pltpu.CompilerParams(use_tc_tiling_on_sc=True),
)
def _sc_embed(idx_hbm, p_hbm, pos_hbm, out_hbm,
              idx_v, pid_v, rows_v, out_v, pos_v, g_sem, w_sem):
    wid = lax.axis_index("s") * NC + lax.axis_index("c")
    pltpu.sync_copy(pos_hbm, pos_v)
    ib = pl.multiple_of(wid * NCH, 8)
    pltpu.sync_copy(idx_hbm.at[pl.ds(ib, NCH)], idx_v)
    out_base = wid * ROWS_PER_W

    def issue_gather(c, b):
        for g in range(CHUNK // L):
            sl = pl.ds(g * L, L)
            ids = idx_v[c, sl]
            pid_v[b, sl] = lax.shift_left(
                lax.shift_right_logical(ids, 9), 8
            ) + jnp.bitwise_and(ids, 255)
        pltpu.async_copy(p_hbm.at[pid_v.at[b]], rows_v.at[b], g_sem.at[b])

    def wait_gather(b):
        pltpu.make_async_copy(
            p_hbm.at[pl.ds(0, CHUNK)], rows_v.at[b], g_sem.at[b]
        ).wait()

    def compact_add(c, b):
        t0 = c * CHUNK

        @plsc.parallel_loop(0, CHUNK // L, 1)
        def _(k):
            r0 = k * L
            cbs = lax.shift_left(
                jnp.bitwise_and(
                    lax.shift_right_logical(idx_v[c, pl.ds(r0, L)], 8), 1
                ),
                6,
            )
            for l in range(L):
                r = r0 + l
                cb = cbs[l]
                q = jnp.bitwise_and(r, 63)
                ob = lax.shift_left(lax.shift_right_logical(r, 6), 6)
                t = lax.rem(t0 + r, T)
                for g in range(D // L):
                    out_v[b, q, pl.ds(ob * D // D + g * L, L)] = (
                        rows_v[b, r, pl.ds(cb + g * L, L)]
                        + pos_v[t, pl.ds(g * L, L)]
                    )

    def issue_wb(c, b):
        off = pl.multiple_of((out_base + c * CHUNK) // 2, CHUNK // 2)
        pltpu.async_copy(
            out_v.at[b], out_hbm.at[pl.ds(off, CHUNK // 2)], w_sem.at[b]
        )

    def wait_wb(b):
        pltpu.make_async_copy(
            out_v.at[b], out_hbm.at[pl.ds(0, CHUNK // 2)], w_sem.at[b]
        ).wait()

    def finish(c, b):
        wait_gather(b)
        compact_add(c, b)
        issue_wb(c, b)

    for c in range(NBUF - 1):
        issue_gather(c, c)

    def outer(i, carry):
        for b2 in range(NBUF):
            c = i * NBUF + b2
            nb = (b2 + NBUF - 1) % NBUF

            @pl.when(c + NBUF - 1 < NCH)
            def _():
                @pl.when(c >= 1)
                def _():
                    wait_wb(nb)

                issue_gather(c + NBUF - 1, nb)

            @pl.when(c < NCH)
            def _():
                finish(c, b2)
        return carry

    lax.fori_loop(0, (NCH + NBUF - 1) // NBUF, outer, 0)
    for b in range(NBUF):
        wait_wb(b)


PACK_COLS = 512
PACK_GRID = -(-V // PACK_COLS)


def _pack_body(tt_ref, out_ref):
    y = tt_ref[...].T
    out_ref[...] = jnp.concatenate(
        [y[0 : PACK_COLS // 2], y[PACK_COLS // 2 : PACK_COLS]], axis=1
    )


_pack_tc = pl.pallas_call(
    _pack_body,
    grid=(PACK_GRID,),
    in_specs=[pl.BlockSpec((D, PACK_COLS), lambda i: (0, i))],
    out_specs=pl.BlockSpec((PACK_COLS // 2, 2 * D), lambda i: (i, 0)),
    out_shape=jax.ShapeDtypeStruct((PACK_GRID * PACK_COLS // 2, 2 * D),
                                    jnp.float32),
)

UNPACK_BB = 16


def _unpack_body(in_ref, out_ref):
    x = in_ref[...]
    lo = x[:, 0:D].reshape(25, 64, D)
    hi = x[:, D : 2 * D].reshape(25, 64, D)
    out_ref[...] = jnp.concatenate([lo, hi], axis=1).reshape(UNPACK_BB, T, D)


_unpack_tc = pl.pallas_call(
    _unpack_body,
    grid=(B // UNPACK_BB,),
    in_specs=[pl.BlockSpec((UNPACK_BB * T // 2, 2 * D), lambda i: (i, 0))],
    out_specs=pl.BlockSpec((UNPACK_BB, T, D), lambda i: (i, 0, 0)),
    out_shape=jax.ShapeDtypeStruct((B, T, D), jnp.float32),
)


def kernel(x, token_table, pos_table):
    idx = x.astype(jnp.int32).reshape(IDX_ROWS, CHUNK)
    pairs = _pack_tc(token_table.T)
    out = _sc_embed(idx, pairs, pos_table)
    return _unpack_tc(out)


# speedup vs baseline: 1.8397x; 1.8397x over previous
import functools

import jax
import jax.numpy as jnp
from jax import lax
from jax.experimental import pallas as pl
from jax.experimental.pallas import tpu as pltpu
from jax.experimental.pallas import tpu_sc as plsc

B, T, D = 4096, 200, 64
ROWS = B * T
NC, NS = 2, 16
NW = NC * NS
ROWS_PER_W = ROWS // NW
GATHER = 100
CHUNK = T
G_PER_CHUNK = CHUNK // GATHER
NCH = ROWS_PER_W // CHUNK
IDX_ROWS = ROWS // GATHER
IDX_PER_W = ROWS_PER_W // GATHER
NBUF = 4

_mesh = plsc.VectorSubcoreMesh(core_axis_name="c", subcore_axis_name="s")


@functools.partial(
    pl.kernel,
    out_type=jax.ShapeDtypeStruct((ROWS, D), jnp.float32),
    mesh=_mesh,
    scratch_types=[
        pltpu.VMEM((IDX_PER_W, GATHER), jnp.int32),
        pltpu.VMEM((NBUF, CHUNK, D), jnp.float32),
        pltpu.VMEM((T, D), jnp.float32),
        pltpu.SemaphoreType.DMA((NBUF,)),
        pltpu.SemaphoreType.DMA((NBUF,)),
    ],
    compiler_params=pltpu.CompilerParams(use_tc_tiling_on_sc=False),
)
def _sc_embed(idx_hbm, table_hbm, pos_hbm, out_hbm, idx_v, rows_v, pos_v,
              g_sem, w_sem):
    wid = lax.axis_index("s") * NC + lax.axis_index("c")
    pltpu.sync_copy(pos_hbm, pos_v)
    pltpu.sync_copy(idx_hbm.at[pl.ds(wid * IDX_PER_W, IDX_PER_W)], idx_v)
    out_base = wid * ROWS_PER_W

    def issue_gather(c, b):
        for j in range(G_PER_CHUNK):
            pltpu.async_copy(
                table_hbm.at[idx_v.at[c * G_PER_CHUNK + j]],
                rows_v.at[b, pl.ds(j * GATHER, GATHER)],
                g_sem.at[b],
            )

    def wait_gathers(b):
        pltpu.make_async_copy(
            out_hbm.at[pl.ds(0, CHUNK)], rows_v.at[b], g_sem.at[b]
        ).wait()

    def add_pos(b):
        @plsc.parallel_loop(0, T, 1, unroll=4)
        def _(r):
            for dd in range(D // 16):
                sl = pl.ds(dd * 16, 16)
                rows_v[b, r, sl] = rows_v[b, r, sl] + pos_v[r, sl]

    def issue_wb(c, b):
        pltpu.async_copy(
            rows_v.at[b], out_hbm.at[pl.ds(out_base + c * CHUNK, CHUNK)],
            w_sem.at[b],
        )

    def wait_wb(b):
        pltpu.make_async_copy(
            rows_v.at[b], out_hbm.at[pl.ds(0, CHUNK)], w_sem.at[b]
        ).wait()

    def finish(c, b):
        wait_gathers(b)
        add_pos(b)
        issue_wb(c, b)

    for c in range(NBUF - 1):
        issue_gather(c, c)
    finish(0, 0)
    issue_gather(NBUF - 1, NBUF - 1)
    for c in range(1, NBUF):
        finish(c, c % NBUF)
        wait_wb((c - 1) % NBUF)
        issue_gather(c + NBUF - 1, (c - 1) % NBUF)

    def outer(i, carry):
        for b2 in range(NBUF):
            c = i * NBUF + b2
            finish(c, b2)
            wait_wb((b2 + NBUF - 1) % NBUF)
            issue_gather(c + NBUF - 1, (b2 + NBUF - 1) % NBUF)
        return carry

    lax.fori_loop(1, NCH // NBUF - 1, outer, 0)

    c0 = NCH - NBUF
    finish(c0, c0 % NBUF)
    wait_wb((c0 - 1) % NBUF)
    issue_gather(NCH - 1, (c0 - 1) % NBUF)
    for c in range(c0 + 1, NCH):
        finish(c, c % NBUF)
    for b in range(NBUF):
        wait_wb(b)


def kernel(x, token_table, pos_table):
    idx = x.astype(jnp.int32).reshape(IDX_ROWS, GATHER)
    out = _sc_embed(idx, token_table, pos_table)
    return out.reshape(B, T, D)
